# baseline (speedup 1.0000x reference)
import jax
import jax.numpy as jnp
from jax.experimental import pallas as pl
from jax.experimental.pallas import tpu as pltpu
from jax.experimental.pallas import tpu_sc as plsc
from jax.experimental.shard_map import shard_map
from jax.sharding import Mesh, PartitionSpec as P

K_CB = 8192
DIM = 64
R = 256
SC_WIN = 128


def _topk_body(x_ref, e_ref, x2_ref, e2_ref, enc_ref, tv_ref, ti_ref):
    x = x_ref[...]
    e = e_ref[...]
    mm = jax.lax.dot_general(
        x, e, (((1,), (1,)), ((), ())), preferred_element_type=jnp.float32
    )
    d = (x2_ref[...] + e2_ref[...]) - 2.0 * mm

    iota = jax.lax.broadcasted_iota(jnp.int32, (R, K_CB), 1).astype(jnp.float32)
    vals = []
    idxs = []
    for k in range(4):
        m = jnp.min(d, axis=1, keepdims=True)
        eq = d == m
        idx = jnp.min(jnp.where(eq, iota, jnp.float32(K_CB)), axis=1, keepdims=True)
        vals.append(m)
        idxs.append(idx)
        if k < 3:
            d = jnp.where(iota == idx, jnp.float32(jnp.inf), d)

    enc_ref[...] = idxs[0].astype(jnp.int32)
    ti_ref[...] = jnp.concatenate(idxs[1:], axis=1).astype(jnp.int32)

    v = jnp.concatenate(vals[1:], axis=1)
    norm = jnp.sqrt(jnp.sum(v * v, axis=1, keepdims=True))
    v = v / jnp.maximum(norm, 1e-12)
    v = 1.0 / (v + 0.0001)
    vmax = jnp.max(v, axis=1, keepdims=True)
    ev = jnp.exp(v - vmax)
    tv_ref[...] = ev / jnp.sum(ev, axis=1, keepdims=True)


def _topk_call(x, e, x2, e2):
    n = x.shape[0]
    return pl.pallas_call(
        _topk_body,
        grid=(n // R,),
        in_specs=[
            pl.BlockSpec((R, DIM), lambda i: (i, 0)),
            pl.BlockSpec((K_CB, DIM), lambda i: (0, 0)),
            pl.BlockSpec((R, 1), lambda i: (i, 0)),
            pl.BlockSpec((1, K_CB), lambda i: (0, 0)),
        ],
        out_specs=[
            pl.BlockSpec((R, 1), lambda i: (i, 0)),
            pl.BlockSpec((R, 3), lambda i: (i, 0)),
            pl.BlockSpec((R, 3), lambda i: (i, 0)),
        ],
        out_shape=[
            jax.ShapeDtypeStruct((n, 1), jnp.int32),
            jax.ShapeDtypeStruct((n, 3), jnp.float32),
            jax.ShapeDtypeStruct((n, 3), jnp.int32),
        ],
    )(x, e, x2, e2)


def _sc_gather(embed_padded, idx_row):
    n = idx_row.shape[1]
    mesh = plsc.VectorSubcoreMesh(core_axis_name="core", subcore_axis_name="subcore")

    @pl.kernel(
        out_type=jax.ShapeDtypeStruct((n, 128), jnp.float32),
        mesh=mesh,
    )
    def gather_kernel(e_hbm, i_hbm, o_hbm):
        def body(i_vmem, o_vmem):
            pltpu.sync_copy(e_hbm.at[i_vmem.at[0]], o_vmem)

        pltpu.emit_pipeline(
            body,
            grid=(n // SC_WIN,),
            in_specs=[pl.BlockSpec((1, SC_WIN), index_map=lambda i: (0, i))],
            out_specs=[pl.BlockSpec((SC_WIN, 128), index_map=lambda i: (i, 0))],
            core_axis_name=("core", "subcore"),
            dimension_semantics=(pltpu.PARALLEL,),
        )(i_hbm, o_hbm)

    return gather_kernel(embed_padded, idx_row)


def _shard_fn(inputs_flatten, embed):
    n = inputs_flatten.shape[0]
    x2 = jnp.sum(inputs_flatten**2, axis=1, keepdims=True)
    e2 = jnp.sum(embed**2, axis=1)[None, :]
    enc_idx, topk_values, topk_indices = _topk_call(inputs_flatten, embed, x2, e2)
    embed_padded = jnp.pad(embed, ((0, 0), (0, 128 - DIM)))
    quantize = _sc_gather(embed_padded, enc_idx.reshape(1, n))[:, :DIM]
    quantize = (quantize - inputs_flatten) + inputs_flatten
    return (quantize, enc_idx, topk_values, topk_indices)


def kernel(inputs_flatten, embed):
    n = inputs_flatten.shape[0]
    devs = jax.devices()
    nd = len(devs)
    while nd > 1 and (n % (nd * R) != 0):
        nd -= 1
    if nd <= 1:
        return _shard_fn(inputs_flatten, embed)
    mesh = Mesh(devs[:nd], ("x",))
    f = shard_map(
        _shard_fn,
        mesh=mesh,
        in_specs=(P("x", None), P(None, None)),
        out_specs=(P("x", None), P("x", None), P("x", None), P("x", None)),
        check_rep=False,
    )
    return f(inputs_flatten, embed)


# speedup vs baseline: 2.1348x; 2.1348x over previous
import jax
import jax.numpy as jnp
from jax.experimental import pallas as pl
from jax.experimental.pallas import tpu as pltpu
from jax.experimental.pallas import tpu_sc as plsc
K_CB = 8192
DIM = 64
R = 512
SC_WIN = 128


def _topk_body(x_ref, e_ref, x2_ref, e2_ref, enc_ref, tv_ref, ti_ref):
    x = x_ref[...]
    e = e_ref[...]
    mm = jax.lax.dot_general(
        x, e, (((1,), (1,)), ((), ())), preferred_element_type=jnp.float32
    )
    d = (x2_ref[...] + e2_ref[...]) - 2.0 * mm

    iota = jax.lax.broadcasted_iota(jnp.int32, (R, K_CB), 1).astype(jnp.float32)
    vals = []
    idxs = []
    for k in range(4):
        m = jnp.min(d, axis=1, keepdims=True)
        eq = d == m
        idx = jnp.min(jnp.where(eq, iota, jnp.float32(K_CB)), axis=1, keepdims=True)
        vals.append(m)
        idxs.append(idx)
        if k < 3:
            d = jnp.where(iota == idx, jnp.float32(jnp.inf), d)

    enc_ref[...] = idxs[0].astype(jnp.int32)
    ti_ref[...] = jnp.concatenate(idxs[1:], axis=1).astype(jnp.int32)

    v = jnp.concatenate(vals[1:], axis=1)
    norm = jnp.sqrt(jnp.sum(v * v, axis=1, keepdims=True))
    v = v / jnp.maximum(norm, 1e-12)
    v = 1.0 / (v + 0.0001)
    vmax = jnp.max(v, axis=1, keepdims=True)
    ev = jnp.exp(v - vmax)
    tv_ref[...] = ev / jnp.sum(ev, axis=1, keepdims=True)


def _topk_call(x, e, x2, e2):
    n = x.shape[0]
    return pl.pallas_call(
        _topk_body,
        grid=(n // R,),
        in_specs=[
            pl.BlockSpec((R, DIM), lambda i: (i, 0)),
            pl.BlockSpec((K_CB, DIM), lambda i: (0, 0)),
            pl.BlockSpec((R, 1), lambda i: (i, 0)),
            pl.BlockSpec((1, K_CB), lambda i: (0, 0)),
        ],
        out_specs=[
            pl.BlockSpec((R, 1), lambda i: (i, 0)),
            pl.BlockSpec((R, 3), lambda i: (i, 0)),
            pl.BlockSpec((R, 3), lambda i: (i, 0)),
        ],
        out_shape=[
            jax.ShapeDtypeStruct((n, 1), jnp.int32),
            jax.ShapeDtypeStruct((n, 3), jnp.float32),
            jax.ShapeDtypeStruct((n, 3), jnp.int32),
        ],
    )(x, e, x2, e2)


def _sc_gather(embed_padded, idx_row):
    n = idx_row.shape[1]
    mesh = plsc.VectorSubcoreMesh(core_axis_name="core", subcore_axis_name="subcore")

    @pl.kernel(
        out_type=jax.ShapeDtypeStruct((n, 128), jnp.float32),
        mesh=mesh,
    )
    def gather_kernel(e_hbm, i_hbm, o_hbm):
        def body(i_vmem, o_vmem):
            pltpu.sync_copy(e_hbm.at[i_vmem.at[0]], o_vmem)

        pltpu.emit_pipeline(
            body,
            grid=(n // SC_WIN,),
            in_specs=[pl.BlockSpec((1, SC_WIN), index_map=lambda i: (0, i))],
            out_specs=[pl.BlockSpec((SC_WIN, 128), index_map=lambda i: (i, 0))],
            core_axis_name=("core", "subcore"),
            dimension_semantics=(pltpu.PARALLEL,),
        )(i_hbm, o_hbm)

    return gather_kernel(embed_padded, idx_row)


def _shard_fn(inputs_flatten, embed):
    n = inputs_flatten.shape[0]
    x2 = jnp.sum(inputs_flatten**2, axis=1, keepdims=True)
    e2 = jnp.sum(embed**2, axis=1)[None, :]
    enc_idx, topk_values, topk_indices = _topk_call(inputs_flatten, embed, x2, e2)
    embed_padded = jnp.pad(embed, ((0, 0), (0, 128 - DIM)))
    quantize = _sc_gather(embed_padded, enc_idx.reshape(1, n))[:, :DIM]
    quantize = (quantize - inputs_flatten) + inputs_flatten
    return (quantize, enc_idx, topk_values, topk_indices)


def kernel(inputs_flatten, embed):
    return _shard_fn(inputs_flatten, embed)
